# baseline (speedup 1.0000x reference)
import functools

import jax
import jax.numpy as jnp
from jax import lax
from jax.experimental import pallas as pl
from jax.experimental.pallas import tpu as pltpu
from jax.experimental.pallas import tpu_sc as plsc

N = 10000
E = 320000
DIMS = [128, 40, 30, 20, 10, 5, 1, 5, 10, 20, 30, 40, 50]
NPAD = 10240
RPS = NPAD // 16
NW = 32
EPW = E // NW
K = 80
NCHUNK = EPW // K

D16 = [-(-DIMS[i + 1] // 16) * 16 for i in range(12)]


def _mesh():
    return plsc.VectorSubcoreMesh(core_axis_name="c", subcore_axis_name="s")


@functools.lru_cache(None)
def _make_msg_prop(d):
    def body(y_hbm, src_hbm, dst_hbm, nrm_hbm, z_hbm, out_hbm,
             acc, sidx, didx, nrm, rows, sem):
        c = lax.axis_index("c")
        s = lax.axis_index("s")
        wid = c * 16 + s
        pltpu.sync_copy(z_hbm.at[pl.ds(s * RPS, RPS)], acc.at[pl.ds(s * RPS, RPS)])
        plsc.subcore_barrier()
        base0 = wid * EPW

        def step(j, carry):
            base = base0 + j * K
            pltpu.sync_copy(src_hbm.at[pl.ds(base, K)], sidx)
            pltpu.sync_copy(dst_hbm.at[pl.ds(base, K)], didx)
            pltpu.sync_copy(nrm_hbm.at[pl.ds(base, K)], nrm)
            pltpu.async_copy(y_hbm.at[sidx], rows, sem).wait()
            for e in range(K):
                nv = nrm[e, pl.ds(0, 16)]
                for t in range(d // 16):
                    sl = pl.ds(t * 16, 16)
                    rows[e, sl] = rows[e, sl] * nv
            pltpu.sync_copy(rows, acc.at[didx], add=True)
            return carry

        lax.fori_loop(0, NCHUNK, step, 0)
        plsc.subcore_barrier()
        pltpu.sync_copy(acc.at[pl.ds(s * RPS, RPS)],
                        out_hbm.at[c].at[pl.ds(s * RPS, RPS)])

    return pl.kernel(
        body,
        out_type=jax.ShapeDtypeStruct((2, NPAD, d), jnp.float32),
        mesh=_mesh(),
        compiler_params=pltpu.CompilerParams(use_tc_tiling_on_sc=False),
        scratch_types=[
            pltpu.VMEM_SHARED((NPAD, d), jnp.float32),
            pltpu.VMEM((K,), jnp.int32),
            pltpu.VMEM((K,), jnp.int32),
            pltpu.VMEM((K, 16), jnp.float32),
            pltpu.VMEM((K, d), jnp.float32),
            pltpu.SemaphoreType.DMA,
        ],
    )


@functools.lru_cache(None)
def _make_deg_prop():
    def body(ones_hbm, dst_hbm, z_hbm, out_hbm, acc, didx, rows, sem):
        c = lax.axis_index("c")
        s = lax.axis_index("s")
        wid = c * 16 + s
        pltpu.sync_copy(z_hbm.at[pl.ds(s * RPS, RPS)], acc.at[pl.ds(s * RPS, RPS)])
        pltpu.sync_copy(ones_hbm, rows)
        plsc.subcore_barrier()
        base0 = wid * EPW

        def step(j, carry):
            base = base0 + j * K
            pltpu.sync_copy(dst_hbm.at[pl.ds(base, K)], didx)
            pltpu.sync_copy(rows, acc.at[didx], add=True)
            return carry

        lax.fori_loop(0, NCHUNK, step, 0)
        plsc.subcore_barrier()
        pltpu.sync_copy(acc.at[pl.ds(s * RPS, RPS)],
                        out_hbm.at[c].at[pl.ds(s * RPS, RPS)])

    return pl.kernel(
        body,
        out_type=jax.ShapeDtypeStruct((2, NPAD, 16), jnp.float32),
        mesh=_mesh(),
        compiler_params=pltpu.CompilerParams(use_tc_tiling_on_sc=False),
        scratch_types=[
            pltpu.VMEM_SHARED((NPAD, 16), jnp.float32),
            pltpu.VMEM((K,), jnp.int32),
            pltpu.VMEM((K, 16), jnp.float32),
            pltpu.SemaphoreType.DMA,
        ],
    )


@functools.lru_cache(None)
def _make_gather16():
    def body(tab_hbm, idx_hbm, out_hbm, idx_v, rows, sem):
        c = lax.axis_index("c")
        s = lax.axis_index("s")
        wid = c * 16 + s
        base0 = wid * EPW

        def step(j, carry):
            base = base0 + j * K
            pltpu.sync_copy(idx_hbm.at[pl.ds(base, K)], idx_v)
            pltpu.async_copy(tab_hbm.at[idx_v], rows, sem).wait()
            pltpu.sync_copy(rows, out_hbm.at[pl.ds(base, K)])
            return carry

        lax.fori_loop(0, NCHUNK, step, 0)

    return pl.kernel(
        body,
        out_type=jax.ShapeDtypeStruct((E, 16), jnp.float32),
        mesh=_mesh(),
        compiler_params=pltpu.CompilerParams(use_tc_tiling_on_sc=False),
        scratch_types=[
            pltpu.VMEM((K,), jnp.int32),
            pltpu.VMEM((K, 16), jnp.float32),
            pltpu.SemaphoreType.DMA,
        ],
    )


@functools.lru_cache(None)
def _make_mm(i):
    din, dpad = DIMS[i], D16[i]

    def body(h_ref, w_ref, o_ref):
        if din == 1:
            o_ref[...] = h_ref[...] * w_ref[...]
        else:
            o_ref[...] = jnp.dot(h_ref[...], w_ref[...],
                                 preferred_element_type=jnp.float32)

    return pl.pallas_call(
        body, out_shape=jax.ShapeDtypeStruct((N, dpad), jnp.float32))


def _bn(h, g, be):
    m = h.mean(axis=0)
    v = h.var(axis=0)
    return (h - m) / jnp.sqrt(v + 1e-5) * g + be


def kernel(x, edge_index, params):
    order = jnp.argsort(edge_index[1], stable=True)
    src = edge_index[0][order]
    dst = edge_index[1][order]
    Ws, bs, gs, bes = params["W"], params["b"], params["g"], params["be"]
    zeros = {d: jnp.zeros((NPAD, d), jnp.float32) for d in sorted(set(D16))}

    ones16 = jnp.ones((K, 16), jnp.float32)
    deg_parts = _make_deg_prop()(ones16, dst, zeros[16])
    deg = 1.0 + (deg_parts[0, :N, 0] + deg_parts[1, :N, 0])
    dinv = jnp.where(deg > 0, 1.0 / jnp.sqrt(deg), 0.0)

    dinv16 = jnp.broadcast_to(dinv[:, None], (N, 16))
    ds_ = _make_gather16()(dinv16, src)[:, 0]
    dd_ = _make_gather16()(dinv16, dst)[:, 0]
    norm = ds_ * dd_
    norm16 = jnp.broadcast_to(norm[:, None], (E, 16))
    dinv2 = dinv * dinv

    h = x
    for i in range(12):
        d = DIMS[i + 1]
        w = jnp.pad(Ws[i], ((0, 0), (0, D16[i] - d)))
        y = _make_mm(i)(h, w)
        parts = _make_msg_prop(D16[i])(y, src, dst, norm16, zeros[D16[i]])
        z = (parts[0, :N, :d] + parts[1, :N, :d]
             + y[:, :d] * dinv2[:, None]) + bs[i]
        if i < 11:
            h = _bn(jax.nn.elu(z), gs[i], bes[i])
        else:
            h = z
    return h
